# baseline (speedup 1.0000x reference)
import functools

import jax
import jax.numpy as jnp
from jax import lax
from jax.experimental import pallas as pl
from jax.experimental.pallas import tpu as pltpu
from jax.experimental.pallas import tpu_sc as plsc

_LANES = 16
_JBLK = 8


@functools.lru_cache(maxsize=None)
def _build(batch, seq_len, vocab, hidden):
    info = plsc.get_sparse_core_info()
    num_workers = info.num_cores * info.num_subcores
    assert batch % num_workers == 0
    blk = batch // num_workers
    assert blk % 8 == 0
    ids_per_worker = blk * seq_len
    assert hidden % (_LANES * _JBLK) == 0
    vecs_per_row = hidden // _LANES

    mesh = plsc.VectorSubcoreMesh(core_axis_name="c", subcore_axis_name="s")

    nbuf = 8
    ahead = 3
    hblk = blk // 2
    nsteps = 2 * seq_len

    def body(
        ids_hbm, table_hbm, pos_hbm, out_hbm, idx_v, pos_v, buf, gsem, psem, ssem
    ):
        wid = lax.axis_index("s") * info.num_cores + lax.axis_index("c")
        b0 = wid * blk
        pltpu.sync_copy(
            ids_hbm.at[pl.ds(wid * ids_per_worker, ids_per_worker)], idx_v
        )

        def gather(t, slot):
            idx_slice = idx_v.at[pl.ds(t * hblk, hblk)]
            return pltpu.make_async_copy(
                table_hbm.at[idx_slice], buf.at[slot], gsem.at[slot]
            )

        def posrow(t, slot):
            return pltpu.make_async_copy(
                pos_hbm.at[lax.div(t, 2)], pos_v.at[slot], psem.at[slot]
            )

        def scatter(t, slot):
            dst = out_hbm.at[
                lax.div(t, 2), pl.ds(b0 + lax.rem(t, 2) * hblk, hblk)
            ]
            return pltpu.make_async_copy(buf.at[slot], dst, ssem.at[slot])

        for t in range(ahead):
            gather(t, t).start()
            posrow(t, t).start()

        @pl.loop(0, nsteps)
        def step_loop(t):
            slot = lax.rem(t, nbuf)
            slot2 = lax.rem(t + ahead, nbuf)

            @pl.when(t >= nbuf - ahead)
            def _():
                scatter(t - (nbuf - ahead), slot2).wait()

            @pl.when(t + ahead < nsteps)
            def _():
                gather(t + ahead, slot2).start()
                posrow(t + ahead, slot2).start()

            gather(t, slot).wait()
            posrow(t, slot).wait()

            for jb in range(vecs_per_row // _JBLK):
                pvs = [
                    pos_v[slot, pl.ds((jb * _JBLK + j) * _LANES, _LANES)]
                    for j in range(_JBLK)
                ]

                @plsc.parallel_loop(0, hblk, unroll=2)
                def row_loop(i, pvs=pvs, jb=jb):
                    for j in range(_JBLK):
                        sl = pl.ds((jb * _JBLK + j) * _LANES, _LANES)
                        plsc.addupdate(buf.at[slot, i, sl], pvs[j])

            scatter(t, slot).start()

        for d in range(nbuf - ahead):
            t = nsteps - (nbuf - ahead) + d
            scatter(t, lax.rem(t, nbuf)).wait()

    return pl.kernel(
        body,
        out_type=jax.ShapeDtypeStruct((seq_len, batch, hidden), jnp.float32),
        mesh=mesh,
        scratch_types=[
            pltpu.VMEM((ids_per_worker,), jnp.int32),
            pltpu.VMEM((nbuf, hidden), jnp.float32),
            pltpu.VMEM((nbuf, hblk, hidden), jnp.float32),
            pltpu.SemaphoreType.DMA((nbuf,)),
            pltpu.SemaphoreType.DMA((nbuf,)),
            pltpu.SemaphoreType.DMA((nbuf,)),
        ],
    )


def kernel(input_ids, token_table, pos_table):
    batch, seq_len = input_ids.shape
    vocab, hidden = token_table.shape
    assert seq_len == pos_table.shape[0]
    info = plsc.get_sparse_core_info()
    num_workers = info.num_cores * info.num_subcores
    blk = batch // num_workers
    ids = (
        input_ids.astype(jnp.int32)
        .reshape(num_workers, blk, seq_len)
        .transpose(0, 2, 1)
        .reshape(-1)
    )
    fn = _build(batch, seq_len, vocab, hidden)
    out = fn(ids, token_table, pos_table)
    return out.transpose(1, 0, 2)


# speedup vs baseline: 1.0996x; 1.0996x over previous
import functools

import jax
import jax.numpy as jnp
from jax import lax
from jax.experimental import pallas as pl
from jax.experimental.pallas import tpu as pltpu
from jax.experimental.pallas import tpu_sc as plsc

_LANES = 16
_JBLK = 8


@functools.lru_cache(maxsize=None)
def _build(batch, seq_len, vocab, hidden):
    info = plsc.get_sparse_core_info()
    num_workers = info.num_cores * info.num_subcores
    assert batch % num_workers == 0
    blk = batch // num_workers
    assert blk % 8 == 0
    ids_per_worker = blk * seq_len
    assert hidden % (_LANES * _JBLK) == 0
    vecs_per_row = hidden // _LANES

    mesh = plsc.VectorSubcoreMesh(core_axis_name="c", subcore_axis_name="s")

    nbuf = 4
    ahead = 2
    cpp = 1
    hblk = blk // cpp
    nsteps = cpp * seq_len

    def body(
        ids_hbm, table_hbm, pos_hbm, out_hbm, idx_v, pos_v, buf, gsem, psem, ssem
    ):
        wid = lax.axis_index("s") * info.num_cores + lax.axis_index("c")
        b0 = wid * blk
        pltpu.sync_copy(
            ids_hbm.at[pl.ds(wid * ids_per_worker, ids_per_worker)], idx_v
        )

        def gather(t, slot):
            idx_slice = idx_v.at[pl.ds(t * hblk, hblk)]
            return pltpu.make_async_copy(
                table_hbm.at[idx_slice], buf.at[slot], gsem.at[slot]
            )

        def posrow(t, slot):
            return pltpu.make_async_copy(
                pos_hbm.at[lax.div(t, cpp)], pos_v.at[slot], psem.at[slot]
            )

        def scatter(t, slot):
            dst = out_hbm.at[
                lax.div(t, cpp), pl.ds(b0 + lax.rem(t, cpp) * hblk, hblk)
            ]
            return pltpu.make_async_copy(buf.at[slot], dst, ssem.at[slot])

        for t in range(ahead):
            gather(t, t).start()
            posrow(t, t).start()

        @pl.loop(0, nsteps)
        def step_loop(t):
            slot = lax.rem(t, nbuf)
            slot2 = lax.rem(t + ahead, nbuf)

            @pl.when(t >= nbuf - ahead)
            def _():
                scatter(t - (nbuf - ahead), slot2).wait()

            @pl.when(t + ahead < nsteps)
            def _():
                gather(t + ahead, slot2).start()
                posrow(t + ahead, slot2).start()

            gather(t, slot).wait()
            posrow(t, slot).wait()

            for jb in range(vecs_per_row // _JBLK):
                pvs = [
                    pos_v[slot, pl.ds((jb * _JBLK + j) * _LANES, _LANES)]
                    for j in range(_JBLK)
                ]

                @plsc.parallel_loop(0, hblk, unroll=2)
                def row_loop(i, pvs=pvs, jb=jb):
                    for j in range(_JBLK):
                        sl = pl.ds((jb * _JBLK + j) * _LANES, _LANES)
                        plsc.addupdate(buf.at[slot, i, sl], pvs[j])

            scatter(t, slot).start()

        for d in range(nbuf - ahead):
            t = nsteps - (nbuf - ahead) + d
            scatter(t, lax.rem(t, nbuf)).wait()

    return pl.kernel(
        body,
        out_type=jax.ShapeDtypeStruct((seq_len, batch, hidden), jnp.float32),
        mesh=mesh,
        scratch_types=[
            pltpu.VMEM((ids_per_worker,), jnp.int32),
            pltpu.VMEM((nbuf, hidden), jnp.float32),
            pltpu.VMEM((nbuf, hblk, hidden), jnp.float32),
            pltpu.SemaphoreType.DMA((nbuf,)),
            pltpu.SemaphoreType.DMA((nbuf,)),
            pltpu.SemaphoreType.DMA((nbuf,)),
        ],
    )


def kernel(input_ids, token_table, pos_table):
    batch, seq_len = input_ids.shape
    vocab, hidden = token_table.shape
    assert seq_len == pos_table.shape[0]
    info = plsc.get_sparse_core_info()
    num_workers = info.num_cores * info.num_subcores
    blk = batch // num_workers
    ids = (
        input_ids.astype(jnp.int32)
        .reshape(num_workers, blk, seq_len)
        .transpose(0, 2, 1)
        .reshape(-1)
    )
    fn = _build(batch, seq_len, vocab, hidden)
    out = fn(ids, token_table, pos_table)
    return out.transpose(1, 0, 2)


# speedup vs baseline: 1.1016x; 1.0018x over previous
import functools

import jax
import jax.numpy as jnp
from jax import lax
from jax.experimental import pallas as pl
from jax.experimental.pallas import tpu as pltpu
from jax.experimental.pallas import tpu_sc as plsc

_LANES = 16
_JBLK = 8


@functools.lru_cache(maxsize=None)
def _build(batch, seq_len, vocab, hidden):
    info = plsc.get_sparse_core_info()
    num_workers = info.num_cores * info.num_subcores
    assert batch % num_workers == 0
    blk = batch // num_workers
    assert blk % 8 == 0
    ids_per_worker = blk * seq_len
    assert hidden % (_LANES * _JBLK) == 0
    vecs_per_row = hidden // _LANES

    mesh = plsc.VectorSubcoreMesh(core_axis_name="c", subcore_axis_name="s")

    nbuf = 4
    ahead = 2
    cpp = 1
    hblk = blk // cpp
    nsteps = cpp * seq_len

    def body(
        ids_hbm, table_hbm, pos_hbm, out_hbm, idx_v, pos_v, buf, gsem, psem, ssem
    ):
        wid = lax.axis_index("s") * info.num_cores + lax.axis_index("c")
        b0 = wid * blk
        pltpu.sync_copy(
            ids_hbm.at[pl.ds(wid * ids_per_worker, ids_per_worker)], idx_v
        )

        def gather(t, slot):
            idx_slice = idx_v.at[pl.ds(t * hblk, hblk)]
            return pltpu.make_async_copy(
                table_hbm.at[idx_slice], buf.at[slot], gsem.at[slot]
            )

        def posrow(t, slot):
            return pltpu.make_async_copy(
                pos_hbm.at[lax.div(t, cpp)], pos_v.at[slot], psem.at[slot]
            )

        def scatter(t, slot):
            dst = out_hbm.at[
                lax.div(t, cpp), pl.ds(b0 + lax.rem(t, cpp) * hblk, hblk)
            ]
            return pltpu.make_async_copy(buf.at[slot], dst, ssem.at[slot])

        for t in range(ahead):
            gather(t, t).start()
            posrow(t, t).start()

        @pl.loop(0, nsteps)
        def step_loop(t):
            slot = lax.rem(t, nbuf)
            slot2 = lax.rem(t + ahead, nbuf)

            @pl.when(t >= nbuf - ahead)
            def _():
                scatter(t - (nbuf - ahead), slot2).wait()

            @pl.when(t + ahead < nsteps)
            def _():
                gather(t + ahead, slot2).start()
                posrow(t + ahead, slot2).start()

            gather(t, slot).wait()
            posrow(t, slot).wait()

            for jb in range(vecs_per_row // _JBLK):
                pvs = [
                    pos_v[slot, pl.ds((jb * _JBLK + j) * _LANES, _LANES)]
                    for j in range(_JBLK)
                ]

                @plsc.parallel_loop(0, hblk, unroll=4)
                def row_loop(i, pvs=pvs, jb=jb):
                    for j in range(_JBLK):
                        sl = pl.ds((jb * _JBLK + j) * _LANES, _LANES)
                        plsc.addupdate(buf.at[slot, i, sl], pvs[j])

            scatter(t, slot).start()

        for d in range(nbuf - ahead):
            t = nsteps - (nbuf - ahead) + d
            scatter(t, lax.rem(t, nbuf)).wait()

    return pl.kernel(
        body,
        out_type=jax.ShapeDtypeStruct((seq_len, batch, hidden), jnp.float32),
        mesh=mesh,
        scratch_types=[
            pltpu.VMEM((ids_per_worker,), jnp.int32),
            pltpu.VMEM((nbuf, hidden), jnp.float32),
            pltpu.VMEM((nbuf, hblk, hidden), jnp.float32),
            pltpu.SemaphoreType.DMA((nbuf,)),
            pltpu.SemaphoreType.DMA((nbuf,)),
            pltpu.SemaphoreType.DMA((nbuf,)),
        ],
    )


def kernel(input_ids, token_table, pos_table):
    batch, seq_len = input_ids.shape
    vocab, hidden = token_table.shape
    assert seq_len == pos_table.shape[0]
    info = plsc.get_sparse_core_info()
    num_workers = info.num_cores * info.num_subcores
    blk = batch // num_workers
    ids = (
        input_ids.astype(jnp.int32)
        .reshape(num_workers, blk, seq_len)
        .transpose(0, 2, 1)
        .reshape(-1)
    )
    fn = _build(batch, seq_len, vocab, hidden)
    out = fn(ids, token_table, pos_table)
    return out.transpose(1, 0, 2)
